# baseline (speedup 1.0000x reference)
import functools

import jax
import jax.numpy as jnp
from jax import lax
from jax.experimental import pallas as pl
from jax.experimental.pallas import tpu as pltpu
from jax.experimental.pallas import tpu_sc as plsc

B = 1024
V = 100000
NS = 16
PER = B // NS
L = 16


def _body(at_hbm, tgt_hbm, out_hbm, tgt_v, rows_v, part_v, sem):
    sid = lax.axis_index("s")
    base = sid * PER

    pltpu.sync_copy(tgt_hbm.at[pl.ds(base, PER)], tgt_v)

    base_c = pl.multiple_of((sid // 2) * 128, 128)
    pltpu.async_copy(at_hbm.at[tgt_v, pl.ds(base_c, 128)], rows_v, sem).wait()

    lanes = lax.broadcasted_iota(jnp.int32, (L,), 0)
    col0 = (sid % 2) * PER
    acc = None
    for j in range(PER // L):
        d = lanes + j * L
        g = plsc.load_gather(rows_v, [d, d + col0])
        acc = g if acc is None else acc + g
    part_v[0] = acc
    pltpu.sync_copy(part_v, out_hbm.at[pl.ds(sid, 1)])


_partials = functools.partial(
    pl.kernel,
    out_type=jax.ShapeDtypeStruct((NS, L), jnp.float32),
    mesh=plsc.VectorSubcoreMesh(core_axis_name="c", subcore_axis_name="s",
                                num_cores=1),
    compiler_params=pltpu.CompilerParams(needs_layout_passes=False),
    scratch_types=[
        pltpu.VMEM((PER,), jnp.int32),
        pltpu.VMEM((PER, 128), jnp.float32),
        pltpu.VMEM((1, L), jnp.float32),
        pltpu.SemaphoreType.DMA,
    ],
)(_body)


def _reduce_body(part_hbm, out_ref, part_v, sem):
    pltpu.make_async_copy(part_hbm, part_v, sem).start()
    pltpu.make_async_copy(part_hbm, part_v, sem).wait()
    out_ref[0, 0] = jnp.sum(part_v[...]) * (-1.0 / B)


_reduce = pl.pallas_call(
    _reduce_body,
    out_shape=jax.ShapeDtypeStruct((1, 1), jnp.float32),
    in_specs=[pl.BlockSpec(memory_space=pl.ANY)],
    out_specs=pl.BlockSpec(memory_space=pltpu.SMEM),
    scratch_shapes=[pltpu.VMEM((NS, L), jnp.float32),
                    pltpu.SemaphoreType.DMA],
)


def kernel(inputs, targets):
    at = inputs.T
    parts = _partials(at, targets.astype(jnp.int32))
    return _reduce(parts)[0, 0]
